# initial kernel scaffold (unmeasured)
import jax
import jax.numpy as jnp
from jax import lax
from jax.experimental import pallas as pl
from jax.experimental.pallas import tpu as pltpu

N_RING = 8
M = 2048
D = 2048
F = 8192
BAND = M // N_RING


def _ring_pos(y, z):
    return jnp.where(y == 0, z, (N_RING - 1) - z).astype(jnp.int32)


def _ring_coords(p):
    y = jnp.where(p < 4, 0, 1).astype(jnp.int32)
    z = jnp.where(p < 4, p, (N_RING - 1) - p).astype(jnp.int32)
    return y, z


def kernel(dy, W):
    my_y = lax.axis_index("y")
    my_z = lax.axis_index("z")
    p_out = _ring_pos(my_y, my_z)
    dy_band = lax.dynamic_slice(dy, (p_out * BAND, 0), (BAND, F)).astype(
        jnp.bfloat16
    )
    w_bf = W.astype(jnp.bfloat16)

    def body(dy_ref, w_ref, out_ref, comm_ref, acc_ref, xrecv_ref,
             send_sems, recv_sems, x_sems):
        my_x = lax.axis_index("x")
        my_y = lax.axis_index("y")
        my_z = lax.axis_index("z")
        p = _ring_pos(my_y, my_z)
        nxt_y, nxt_z = _ring_coords(lax.rem(p + 1, N_RING))
        prv_y, prv_z = _ring_coords(lax.rem(p + N_RING - 1, N_RING))

        barrier = pltpu.get_barrier_semaphore()
        for tgt in [(1 - my_x, my_y, my_z),
                    (my_x, nxt_y, nxt_z),
                    (my_x, prv_y, prv_z)]:
            pl.semaphore_signal(barrier, inc=1, device_id=tgt,
                                device_id_type=pl.DeviceIdType.MESH)
        pl.semaphore_wait(barrier, 3)

        acc_ref[...] = lax.dot_general(
            dy_ref[...], w_ref[...],
            dimension_numbers=(((1,), (1,)), ((), ())),
            preferred_element_type=jnp.float32,
        )

        xr = pltpu.make_async_remote_copy(
            src_ref=acc_ref,
            dst_ref=xrecv_ref,
            send_sem=x_sems.at[0],
            recv_sem=x_sems.at[1],
            device_id=(1 - my_x, my_y, my_z),
            device_id_type=pl.DeviceIdType.MESH,
        )
        xr.start()
        xr.wait()
        final = acc_ref[...] + xrecv_ref[...]
        comm_ref[0, :, :] = final
        out_ref[pl.ds(p * BAND, BAND), :] = final

        for h in range(N_RING - 1):
            rdma = pltpu.make_async_remote_copy(
                src_ref=comm_ref.at[h],
                dst_ref=comm_ref.at[h + 1],
                send_sem=send_sems.at[h],
                recv_sem=recv_sems.at[h + 1],
                device_id=(my_x, nxt_y, nxt_z),
                device_id_type=pl.DeviceIdType.MESH,
            )
            rdma.start()
            rdma.wait()
            origin = lax.rem(p + N_RING - h - 1, N_RING)
            out_ref[pl.ds(origin * BAND, BAND), :] = comm_ref[h + 1, :, :]

    return pl.pallas_call(
        body,
        out_shape=jax.ShapeDtypeStruct((M, D), jnp.float32),
        in_specs=[
            pl.BlockSpec(memory_space=pltpu.VMEM),
            pl.BlockSpec(memory_space=pltpu.VMEM),
        ],
        out_specs=pl.BlockSpec(memory_space=pltpu.VMEM),
        scratch_shapes=[
            pltpu.VMEM((N_RING, BAND, D), jnp.float32),
            pltpu.VMEM((BAND, D), jnp.float32),
            pltpu.VMEM((BAND, D), jnp.float32),
            pltpu.SemaphoreType.DMA((N_RING,)),
            pltpu.SemaphoreType.DMA((N_RING,)),
            pltpu.SemaphoreType.DMA((2,)),
        ],
        compiler_params=pltpu.CompilerParams(collective_id=0),
    )(dy_band, w_bf)


# baseline (device time: 276206 ns/iter reference)
import jax
import jax.numpy as jnp
from jax import lax
from jax.experimental import pallas as pl
from jax.experimental.pallas import tpu as pltpu

N_RING = 8
M = 2048
D = 2048
F = 8192
BAND = M // N_RING


def _ring_pos(y, z):
    return jnp.where(y == 0, z, (N_RING - 1) - z).astype(jnp.int32)


def _ring_coords(p):
    y = jnp.where(p < 4, 0, 1).astype(jnp.int32)
    z = jnp.where(p < 4, p, (N_RING - 1) - p).astype(jnp.int32)
    return y, z


def kernel(dy, W):
    my_y = lax.axis_index("y")
    my_z = lax.axis_index("z")
    p_out = _ring_pos(my_y, my_z)
    dy_band = lax.dynamic_slice(dy, (p_out * BAND, 0), (BAND, F)).astype(
        jnp.bfloat16
    )
    w_bf = W.astype(jnp.bfloat16)

    def body(dy_ref, w_ref, out_ref, acc_ref, xrecv_ref,
             send_sems, recv_sems, x_sems):
        my_x = lax.axis_index("x")
        my_y = lax.axis_index("y")
        my_z = lax.axis_index("z")
        p = _ring_pos(my_y, my_z)
        nxt_y, nxt_z = _ring_coords(lax.rem(p + 1, N_RING))
        prv_y, prv_z = _ring_coords(lax.rem(p + N_RING - 1, N_RING))

        barrier = pltpu.get_barrier_semaphore()
        for tgt in [(1 - my_x, my_y, my_z),
                    (my_x, nxt_y, nxt_z),
                    (my_x, prv_y, prv_z)]:
            pl.semaphore_signal(barrier, inc=1, device_id=tgt,
                                device_id_type=pl.DeviceIdType.MESH)
        pl.semaphore_wait(barrier, 3)

        acc_ref[...] = lax.dot_general(
            dy_ref[...], w_ref[...],
            dimension_numbers=(((1,), (1,)), ((), ())),
            preferred_element_type=jnp.float32,
        )

        xr = pltpu.make_async_remote_copy(
            src_ref=acc_ref,
            dst_ref=xrecv_ref,
            send_sem=x_sems.at[0],
            recv_sem=x_sems.at[1],
            device_id=(1 - my_x, my_y, my_z),
            device_id_type=pl.DeviceIdType.MESH,
        )
        xr.start()
        xr.wait()
        out_ref[pl.ds(p * BAND, BAND), :] = acc_ref[...] + xrecv_ref[...]

        for h in range(N_RING - 1):
            origin = lax.rem(p + N_RING - h, N_RING)
            rdma = pltpu.make_async_remote_copy(
                src_ref=out_ref.at[pl.ds(origin * BAND, BAND), :],
                dst_ref=out_ref.at[pl.ds(origin * BAND, BAND), :],
                send_sem=send_sems.at[h],
                recv_sem=recv_sems.at[h],
                device_id=(my_x, nxt_y, nxt_z),
                device_id_type=pl.DeviceIdType.MESH,
            )
            rdma.start()
            rdma.wait()

    return pl.pallas_call(
        body,
        out_shape=jax.ShapeDtypeStruct((M, D), jnp.float32),
        in_specs=[
            pl.BlockSpec(memory_space=pltpu.VMEM),
            pl.BlockSpec(memory_space=pltpu.VMEM),
        ],
        out_specs=pl.BlockSpec(memory_space=pltpu.VMEM),
        scratch_shapes=[
            pltpu.VMEM((BAND, D), jnp.float32),
            pltpu.VMEM((BAND, D), jnp.float32),
            pltpu.SemaphoreType.DMA((N_RING,)),
            pltpu.SemaphoreType.DMA((N_RING,)),
            pltpu.SemaphoreType.DMA((2,)),
        ],
        compiler_params=pltpu.CompilerParams(
            collective_id=0,
            vmem_limit_bytes=110 * 1024 * 1024,
        ),
    )(dy_band, w_bf)


# device time: 134980 ns/iter; 2.0463x vs baseline; 2.0463x over previous
import jax
import jax.numpy as jnp
from jax import lax
from jax.experimental import pallas as pl
from jax.experimental.pallas import tpu as pltpu

N_RING = 8
M = 2048
D = 2048
F = 8192
BAND = M // N_RING
HALF = D // 2
N_HOP = 4


def _ring_pos(y, z):
    return jnp.where(y == 0, z, (N_RING - 1) - z).astype(jnp.int32)


def _ring_coords(p):
    y = jnp.where(p < 4, 0, 1).astype(jnp.int32)
    z = jnp.where(p < 4, p, (N_RING - 1) - p).astype(jnp.int32)
    return y, z


def kernel(dy, W):
    my_y = lax.axis_index("y")
    my_z = lax.axis_index("z")
    p_out = _ring_pos(my_y, my_z)
    dy_band = lax.dynamic_slice(dy, (p_out * BAND, 0), (BAND, F)).astype(
        jnp.bfloat16
    )
    w_bf = W.astype(jnp.bfloat16)

    def body(dy_ref, w_ref, out_ref, acc_ref, xsend_ref, xrecv_ref,
             cw_send, cw_recv, ccw_send, ccw_recv, x_sems):
        my_x = lax.axis_index("x")
        my_y = lax.axis_index("y")
        my_z = lax.axis_index("z")
        p = _ring_pos(my_y, my_z)
        nxt_y, nxt_z = _ring_coords(lax.rem(p + 1, N_RING))
        prv_y, prv_z = _ring_coords(lax.rem(p + N_RING - 1, N_RING))
        nxt = (my_x, nxt_y, nxt_z)
        prv = (my_x, prv_y, prv_z)

        barrier = pltpu.get_barrier_semaphore()
        for tgt in [(1 - my_x, my_y, my_z), nxt, prv]:
            pl.semaphore_signal(barrier, inc=1, device_id=tgt,
                                device_id_type=pl.DeviceIdType.MESH)
        pl.semaphore_wait(barrier, 3)

        acc_ref[...] = lax.dot_general(
            dy_ref[...], w_ref[...],
            dimension_numbers=(((1,), (1,)), ((), ())),
            preferred_element_type=jnp.float32,
        )

        xsend_ref[...] = acc_ref[...].astype(jnp.bfloat16)
        xr = pltpu.make_async_remote_copy(
            src_ref=xsend_ref,
            dst_ref=xrecv_ref,
            send_sem=x_sems.at[0],
            recv_sem=x_sems.at[1],
            device_id=(1 - my_x, my_y, my_z),
            device_id_type=pl.DeviceIdType.MESH,
        )
        xr.start()
        xr.wait()
        out_ref[pl.ds(p * BAND, BAND), :] = (
            acc_ref[...] + xrecv_ref[...].astype(jnp.float32)
        ).astype(jnp.bfloat16)

        for h in range(N_HOP):
            cw_b = lax.rem(p + N_RING - h, N_RING)
            ccw_b = lax.rem(p + h, N_RING)
            if h < 3:
                cw_cols = ccw_cols = slice(None)
            else:
                cw_cols = pl.ds(0, HALF)
                ccw_cols = pl.ds(HALF, HALF)
            cw = pltpu.make_async_remote_copy(
                src_ref=out_ref.at[pl.ds(cw_b * BAND, BAND), cw_cols],
                dst_ref=out_ref.at[pl.ds(cw_b * BAND, BAND), cw_cols],
                send_sem=cw_send.at[h],
                recv_sem=cw_recv.at[h],
                device_id=nxt,
                device_id_type=pl.DeviceIdType.MESH,
            )
            ccw = pltpu.make_async_remote_copy(
                src_ref=out_ref.at[pl.ds(ccw_b * BAND, BAND), ccw_cols],
                dst_ref=out_ref.at[pl.ds(ccw_b * BAND, BAND), ccw_cols],
                send_sem=ccw_send.at[h],
                recv_sem=ccw_recv.at[h],
                device_id=prv,
                device_id_type=pl.DeviceIdType.MESH,
            )
            cw.start()
            ccw.start()
            cw.wait()
            ccw.wait()

    return pl.pallas_call(
        body,
        out_shape=jax.ShapeDtypeStruct((M, D), jnp.bfloat16),
        in_specs=[
            pl.BlockSpec(memory_space=pltpu.VMEM),
            pl.BlockSpec(memory_space=pltpu.VMEM),
        ],
        out_specs=pl.BlockSpec(memory_space=pltpu.VMEM),
        scratch_shapes=[
            pltpu.VMEM((BAND, D), jnp.float32),
            pltpu.VMEM((BAND, D), jnp.bfloat16),
            pltpu.VMEM((BAND, D), jnp.bfloat16),
            pltpu.SemaphoreType.DMA((N_HOP,)),
            pltpu.SemaphoreType.DMA((N_HOP,)),
            pltpu.SemaphoreType.DMA((N_HOP,)),
            pltpu.SemaphoreType.DMA((N_HOP,)),
            pltpu.SemaphoreType.DMA((2,)),
        ],
        compiler_params=pltpu.CompilerParams(
            collective_id=0,
            vmem_limit_bytes=110 * 1024 * 1024,
        ),
    )(dy_band, w_bf)


# device time: 83776 ns/iter; 3.2970x vs baseline; 1.6112x over previous
import jax
import jax.numpy as jnp
from jax import lax
from jax.experimental import pallas as pl
from jax.experimental.pallas import tpu as pltpu

N_RING = 8
M = 2048
D = 2048
F = 8192
BR = 1024
BC = 512
SR = BR // 2
CHUNK = 128
N_HOP = 4


def _ring_pos(y, z):
    return jnp.where(y == 0, z, (N_RING - 1) - z).astype(jnp.int32)


def _ring_coords(p):
    y = jnp.where(p < 4, 0, 1).astype(jnp.int32)
    z = jnp.where(p < 4, p, (N_RING - 1) - p).astype(jnp.int32)
    return y, z


def _sub_ref(out_ref, q, s, half=None):
    yq, zq = _ring_coords(q)
    rows = pl.ds(yq * BR + s * SR, SR)
    if half is None:
        cols = pl.ds(zq * BC, BC)
    else:
        cols = pl.ds(zq * BC + half * (BC // 2), BC // 2)
    return out_ref.at[rows, cols]


def kernel(dy, W):
    def body(dy_ref, w_ref, out_ref, dy_bf, w_bf, stage, acc_ref,
             xsend_ref, xrecv_ref, load_sems,
             cw_send, cw_recv, ccw_send, ccw_recv, x_sems):
        my_x = lax.axis_index("x")
        my_y = lax.axis_index("y")
        my_z = lax.axis_index("z")
        p = _ring_pos(my_y, my_z)
        nxt_y, nxt_z = _ring_coords(lax.rem(p + 1, N_RING))
        prv_y, prv_z = _ring_coords(lax.rem(p + N_RING - 1, N_RING))
        nxt = (my_x, nxt_y, nxt_z)
        prv = (my_x, prv_y, prv_z)

        chunks = (
            [(w_ref, my_z * BC + i * CHUNK, w_bf, i * CHUNK)
             for i in range(BC // CHUNK)]
            + [(dy_ref, my_y * BR + i * CHUNK, dy_bf, i * CHUNK)
               for i in range(BR // CHUNK)]
        )
        n_a = BC // CHUNK + SR // CHUNK
        n_all = len(chunks)

        def start_chunk(i):
            src, off, _, _ = chunks[i]
            pltpu.make_async_copy(
                src.at[pl.ds(off, CHUNK), :],
                stage.at[i % 2],
                load_sems.at[i % 2],
            ).start()

        def finish_chunk(i):
            src, off, dst, doff = chunks[i]
            pltpu.make_async_copy(
                src.at[pl.ds(off, CHUNK), :],
                stage.at[i % 2],
                load_sems.at[i % 2],
            ).wait()
            dst[pl.ds(doff, CHUNK), :] = stage[i % 2].astype(jnp.bfloat16)

        start_chunk(0)
        for i in range(n_a):
            if i + 1 < n_all:
                start_chunk(i + 1)
            finish_chunk(i)

        def gemm(s):
            acc_ref[pl.ds(s * SR, SR), :] = lax.dot_general(
                dy_bf[pl.ds(s * SR, SR), :], w_bf[...],
                dimension_numbers=(((1,), (1,)), ((), ())),
                preferred_element_type=jnp.float32,
            )

        gemm(0)

        barrier = pltpu.get_barrier_semaphore()
        for tgt in [(1 - my_x, my_y, my_z), nxt, prv]:
            pl.semaphore_signal(barrier, inc=1, device_id=tgt,
                                device_id_type=pl.DeviceIdType.MESH)
        pl.semaphore_wait(barrier, 3)

        def make_xchg(s):
            return pltpu.make_async_remote_copy(
                src_ref=xsend_ref.at[pl.ds(s * SR, SR), :],
                dst_ref=xrecv_ref.at[pl.ds(s * SR, SR), :],
                send_sem=x_sems.at[s, 0],
                recv_sem=x_sems.at[s, 1],
                device_id=(1 - my_x, my_y, my_z),
                device_id_type=pl.DeviceIdType.MESH,
            )

        xsend_ref[pl.ds(0, SR), :] = acc_ref[pl.ds(0, SR), :].astype(
            jnp.bfloat16)
        xchg0 = make_xchg(0)
        xchg0.start()

        for i in range(n_a, n_all):
            if i + 1 < n_all:
                start_chunk(i + 1)
            finish_chunk(i)
        gemm(1)
        xsend_ref[pl.ds(SR, SR), :] = acc_ref[pl.ds(SR, SR), :].astype(
            jnp.bfloat16)

        def finalize(s):
            rows = pl.ds(s * SR, SR)
            out_ref[pl.ds(my_y * BR + s * SR, SR),
                    pl.ds(my_z * BC, BC)] = (
                acc_ref[rows, :] + xrecv_ref[rows, :].astype(jnp.float32)
            ).astype(jnp.bfloat16)

        def make_hop(s, h):
            cw_b = lax.rem(p + N_RING - h, N_RING)
            ccw_b = lax.rem(p + h, N_RING)
            cw_half = None if h < 3 else 0
            ccw_half = None if h < 3 else 1
            cw = pltpu.make_async_remote_copy(
                src_ref=_sub_ref(out_ref, cw_b, s, cw_half),
                dst_ref=_sub_ref(out_ref, cw_b, s, cw_half),
                send_sem=cw_send.at[s, h],
                recv_sem=cw_recv.at[s, h],
                device_id=nxt,
                device_id_type=pl.DeviceIdType.MESH,
            )
            ccw = pltpu.make_async_remote_copy(
                src_ref=_sub_ref(out_ref, ccw_b, s, ccw_half),
                dst_ref=_sub_ref(out_ref, ccw_b, s, ccw_half),
                send_sem=ccw_send.at[s, h],
                recv_sem=ccw_recv.at[s, h],
                device_id=prv,
                device_id_type=pl.DeviceIdType.MESH,
            )
            return cw, ccw

        def start_hop(hop):
            hop[0].start()
            hop[1].start()

        def wait_hop(hop):
            hop[0].wait()
            hop[1].wait()

        xchg1 = make_xchg(1)
        xchg1.start()
        xchg0.wait()
        finalize(0)

        r0 = [make_hop(0, h) for h in range(N_HOP)]
        r1 = [make_hop(1, h) for h in range(N_HOP)]

        start_hop(r0[0])
        xchg1.wait()
        finalize(1)
        start_hop(r1[0])
        wait_hop(r0[0])
        start_hop(r0[1])
        wait_hop(r1[0])
        start_hop(r1[1])
        wait_hop(r0[1])
        start_hop(r0[2])
        wait_hop(r1[1])
        start_hop(r1[2])
        wait_hop(r0[2])
        start_hop(r0[3])
        wait_hop(r1[2])
        start_hop(r1[3])
        wait_hop(r0[3])
        wait_hop(r1[3])

    return pl.pallas_call(
        body,
        out_shape=jax.ShapeDtypeStruct((M, D), jnp.bfloat16),
        in_specs=[
            pl.BlockSpec(memory_space=pl.ANY),
            pl.BlockSpec(memory_space=pl.ANY),
        ],
        out_specs=pl.BlockSpec(memory_space=pltpu.VMEM),
        scratch_shapes=[
            pltpu.VMEM((BR, F), jnp.bfloat16),
            pltpu.VMEM((BC, F), jnp.bfloat16),
            pltpu.VMEM((2, CHUNK, F), jnp.float32),
            pltpu.VMEM((BR, BC), jnp.float32),
            pltpu.VMEM((BR, BC), jnp.bfloat16),
            pltpu.VMEM((BR, BC), jnp.bfloat16),
            pltpu.SemaphoreType.DMA((2,)),
            pltpu.SemaphoreType.DMA((2, N_HOP)),
            pltpu.SemaphoreType.DMA((2, N_HOP)),
            pltpu.SemaphoreType.DMA((2, N_HOP)),
            pltpu.SemaphoreType.DMA((2, N_HOP)),
            pltpu.SemaphoreType.DMA((2, 2)),
        ],
        compiler_params=pltpu.CompilerParams(
            collective_id=0,
            vmem_limit_bytes=110 * 1024 * 1024,
        ),
    )(dy, W)


# device time: 79523 ns/iter; 3.4733x vs baseline; 1.0535x over previous
import jax
import jax.numpy as jnp
from jax import lax
from jax.experimental import pallas as pl
from jax.experimental.pallas import tpu as pltpu

N_RING = 8
M = 2048
D = 2048
F = 8192
BR = 1024
BC = 512
SR = BR // 2
Q = BC // 4
CHUNK = 128
N_HOP = 4
N_MIR = 7


def _ring_pos(y, z):
    return jnp.where(y == 0, z, (N_RING - 1) - z).astype(jnp.int32)


def _ring_coords(p):
    y = jnp.where(p < 4, 0, 1).astype(jnp.int32)
    z = jnp.where(p < 4, p, (N_RING - 1) - p).astype(jnp.int32)
    return y, z


def _sub_ref(out_ref, q, s, coff, width):
    yq, zq = _ring_coords(q)
    return out_ref.at[pl.ds(yq * BR + s * SR, SR),
                      pl.ds(zq * BC + coff, width)]


def kernel(dy, W):
    def body(dy_ref, w_ref, out_ref, dy_bf, w_bf, stage, acc_ref,
             xsend_ref, xrecv_ref, load_sems,
             cw_send, cw_recv, ccw_send, ccw_recv, x_sems,
             m_send, m_recv):
        my_x = lax.axis_index("x")
        my_y = lax.axis_index("y")
        my_z = lax.axis_index("z")
        p = _ring_pos(my_y, my_z)
        nxt_y, nxt_z = _ring_coords(lax.rem(p + 1, N_RING))
        prv_y, prv_z = _ring_coords(lax.rem(p + N_RING - 1, N_RING))
        nxt = (my_x, nxt_y, nxt_z)
        prv = (my_x, prv_y, prv_z)
        xpeer = (1 - my_x, my_y, my_z)
        rcoff = my_x * Q
        mcoff = my_x * 3 * Q

        chunks = (
            [(w_ref, my_z * BC + i * CHUNK, w_bf, i * CHUNK)
             for i in range(BC // CHUNK)]
            + [(dy_ref, my_y * BR + i * CHUNK, dy_bf, i * CHUNK)
               for i in range(BR // CHUNK)]
        )
        n_a = BC // CHUNK + SR // CHUNK
        n_all = len(chunks)

        def start_chunk(i):
            src, off, _, _ = chunks[i]
            pltpu.make_async_copy(
                src.at[pl.ds(off, CHUNK), :],
                stage.at[i % 2],
                load_sems.at[i % 2],
            ).start()

        def finish_chunk(i):
            src, off, dst, doff = chunks[i]
            pltpu.make_async_copy(
                src.at[pl.ds(off, CHUNK), :],
                stage.at[i % 2],
                load_sems.at[i % 2],
            ).wait()
            dst[pl.ds(doff, CHUNK), :] = stage[i % 2].astype(jnp.bfloat16)

        start_chunk(0)
        for i in range(n_a):
            if i + 1 < n_all:
                start_chunk(i + 1)
            finish_chunk(i)

        def gemm(s):
            acc_ref[pl.ds(s * SR, SR), :] = lax.dot_general(
                dy_bf[pl.ds(s * SR, SR), :], w_bf[...],
                dimension_numbers=(((1,), (1,)), ((), ())),
                preferred_element_type=jnp.float32,
            )

        gemm(0)

        barrier = pltpu.get_barrier_semaphore()
        for tgt in [xpeer, nxt, prv]:
            pl.semaphore_signal(barrier, inc=1, device_id=tgt,
                                device_id_type=pl.DeviceIdType.MESH)
        pl.semaphore_wait(barrier, 3)

        def make_xchg(s):
            return pltpu.make_async_remote_copy(
                src_ref=xsend_ref.at[pl.ds(s * SR, SR), :],
                dst_ref=xrecv_ref.at[pl.ds(s * SR, SR), :],
                send_sem=x_sems.at[s, 0],
                recv_sem=x_sems.at[s, 1],
                device_id=xpeer,
                device_id_type=pl.DeviceIdType.MESH,
            )

        xsend_ref[pl.ds(0, SR), :] = acc_ref[pl.ds(0, SR), :].astype(
            jnp.bfloat16)
        xchg0 = make_xchg(0)
        xchg0.start()

        for i in range(n_a, n_all):
            if i + 1 < n_all:
                start_chunk(i + 1)
            finish_chunk(i)
        gemm(1)
        xsend_ref[pl.ds(SR, SR), :] = acc_ref[pl.ds(SR, SR), :].astype(
            jnp.bfloat16)

        def finalize(s):
            rows = pl.ds(s * SR, SR)
            out_ref[pl.ds(my_y * BR + s * SR, SR),
                    pl.ds(my_z * BC, BC)] = (
                acc_ref[rows, :] + xrecv_ref[rows, :].astype(jnp.float32)
            ).astype(jnp.bfloat16)

        def make_hop(s, h):
            cw_b = lax.rem(p + N_RING - h, N_RING)
            ccw_b = lax.rem(p + h, N_RING)
            if h < 3:
                cw_co, cw_w = rcoff, 3 * Q
                ccw_co, ccw_w = rcoff, 3 * Q
            else:
                cw_co, cw_w = rcoff, 2 * Q
                ccw_co, ccw_w = rcoff + 2 * Q, Q
            cw = pltpu.make_async_remote_copy(
                src_ref=_sub_ref(out_ref, cw_b, s, cw_co, cw_w),
                dst_ref=_sub_ref(out_ref, cw_b, s, cw_co, cw_w),
                send_sem=cw_send.at[s, h],
                recv_sem=cw_recv.at[s, h],
                device_id=nxt,
                device_id_type=pl.DeviceIdType.MESH,
            )
            ccw = pltpu.make_async_remote_copy(
                src_ref=_sub_ref(out_ref, ccw_b, s, ccw_co, ccw_w),
                dst_ref=_sub_ref(out_ref, ccw_b, s, ccw_co, ccw_w),
                send_sem=ccw_send.at[s, h],
                recv_sem=ccw_recv.at[s, h],
                device_id=prv,
                device_id_type=pl.DeviceIdType.MESH,
            )
            return cw, ccw

        def start_hop(hop):
            hop[0].start()
            hop[1].start()

        def wait_hop(hop):
            hop[0].wait()
            hop[1].wait()

        mirrors = []

        def mirror(s, j, blk):
            m = pltpu.make_async_remote_copy(
                src_ref=_sub_ref(out_ref, blk, s, mcoff, Q),
                dst_ref=_sub_ref(out_ref, blk, s, mcoff, Q),
                send_sem=m_send.at[s, j],
                recv_sem=m_recv.at[s, j],
                device_id=xpeer,
                device_id_type=pl.DeviceIdType.MESH,
            )
            m.start()
            mirrors.append(m)

        def mirror_hop(s, h):
            if h < 3:
                mirror(s, 2 * h, lax.rem(p + N_RING - h - 1, N_RING))
                mirror(s, 2 * h + 1, lax.rem(p + h + 1, N_RING))
            else:
                mirror(s, 6, lax.rem(p + N_HOP, N_RING))

        xchg1 = make_xchg(1)
        xchg1.start()
        xchg0.wait()
        finalize(0)

        r0 = [make_hop(0, h) for h in range(N_HOP)]
        r1 = [make_hop(1, h) for h in range(N_HOP)]

        start_hop(r0[0])
        xchg1.wait()
        finalize(1)
        start_hop(r1[0])
        wait_hop(r0[0])
        start_hop(r0[1])
        mirror_hop(0, 0)
        wait_hop(r1[0])
        start_hop(r1[1])
        mirror_hop(1, 0)
        wait_hop(r0[1])
        start_hop(r0[2])
        mirror_hop(0, 1)
        wait_hop(r1[1])
        start_hop(r1[2])
        mirror_hop(1, 1)
        wait_hop(r0[2])
        start_hop(r0[3])
        mirror_hop(0, 2)
        wait_hop(r1[2])
        start_hop(r1[3])
        mirror_hop(1, 2)
        wait_hop(r0[3])
        mirror_hop(0, 3)
        wait_hop(r1[3])
        mirror_hop(1, 3)
        for m in mirrors:
            m.wait()

    return pl.pallas_call(
        body,
        out_shape=jax.ShapeDtypeStruct((M, D), jnp.bfloat16),
        in_specs=[
            pl.BlockSpec(memory_space=pl.ANY),
            pl.BlockSpec(memory_space=pl.ANY),
        ],
        out_specs=pl.BlockSpec(memory_space=pltpu.VMEM),
        scratch_shapes=[
            pltpu.VMEM((BR, F), jnp.bfloat16),
            pltpu.VMEM((BC, F), jnp.bfloat16),
            pltpu.VMEM((2, CHUNK, F), jnp.float32),
            pltpu.VMEM((BR, BC), jnp.float32),
            pltpu.VMEM((BR, BC), jnp.bfloat16),
            pltpu.VMEM((BR, BC), jnp.bfloat16),
            pltpu.SemaphoreType.DMA((2,)),
            pltpu.SemaphoreType.DMA((2, N_HOP)),
            pltpu.SemaphoreType.DMA((2, N_HOP)),
            pltpu.SemaphoreType.DMA((2, N_HOP)),
            pltpu.SemaphoreType.DMA((2, N_HOP)),
            pltpu.SemaphoreType.DMA((2, 2)),
            pltpu.SemaphoreType.DMA((2, N_MIR)),
            pltpu.SemaphoreType.DMA((2, N_MIR)),
        ],
        compiler_params=pltpu.CompilerParams(
            collective_id=0,
            vmem_limit_bytes=110 * 1024 * 1024,
        ),
    )(dy, W)


# device time: 71113 ns/iter; 3.8840x vs baseline; 1.1183x over previous
import jax
import jax.numpy as jnp
from jax import lax
from jax.experimental import pallas as pl
from jax.experimental.pallas import tpu as pltpu

N_RING = 8
M = 2048
D = 2048
F = 8192
BR = 1024
BC = 512
NS = 4
SR = BR // NS
RR = 3 * SR // 4
MS = SR // 4
CHUNK = 128
N_STG = 3
N_HOP = 4
N_MIR = 7


def _ring_pos(y, z):
    return jnp.where(y == 0, z, (N_RING - 1) - z).astype(jnp.int32)


def _ring_coords(p):
    y = jnp.where(p < 4, 0, 1).astype(jnp.int32)
    z = jnp.where(p < 4, p, (N_RING - 1) - p).astype(jnp.int32)
    return y, z


def kernel(dy, W):
    def body(dy_ref, w_ref, out_ref, dy_bf, w_bf, stage, acc_ref,
             xsend_ref, xrecv_ref, own_stg, cwb, ccwb, mbuf, load_sems,
             cw_send, cw_recv, ccw_send, ccw_recv, x_sems,
             m_send, m_recv):
        my_x = lax.axis_index("x")
        my_y = lax.axis_index("y")
        my_z = lax.axis_index("z")
        p = _ring_pos(my_y, my_z)
        nxt_y, nxt_z = _ring_coords(lax.rem(p + 1, N_RING))
        prv_y, prv_z = _ring_coords(lax.rem(p + N_RING - 1, N_RING))
        nxt = (my_x, nxt_y, nxt_z)
        prv = (my_x, prv_y, prv_z)
        xpeer = (1 - my_x, my_y, my_z)
        rroff = my_x * MS
        mrel = my_x * 2 * MS

        def blk_rows(q, s, rel, height):
            yq, zq = _ring_coords(q)
            return (pl.ds(yq * BR + s * SR + rel, height),
                    pl.ds(zq * BC, BC))

        chunks = (
            [(w_ref, my_z * BC + i * CHUNK, w_bf, i * CHUNK)
             for i in range(BC // CHUNK)]
            + [(dy_ref, my_y * BR + i * CHUNK, dy_bf, i * CHUNK)
               for i in range(BR // CHUNK)]
        )
        n_w = BC // CHUNK
        per_s = SR // CHUNK
        n_all = len(chunks)

        def start_chunk(i):
            src, off, _, _ = chunks[i]
            pltpu.make_async_copy(
                src.at[pl.ds(off, CHUNK), :],
                stage.at[i % N_STG],
                load_sems.at[i % N_STG],
            ).start()

        def finish_chunk(i):
            src, off, dst, doff = chunks[i]
            pltpu.make_async_copy(
                src.at[pl.ds(off, CHUNK), :],
                stage.at[i % N_STG],
                load_sems.at[i % N_STG],
            ).wait()
            dst[pl.ds(doff, CHUNK), :] = stage[i % N_STG].astype(
                jnp.bfloat16)

        def finish_upto(lo, hi):
            for i in range(lo, hi):
                finish_chunk(i)
                if i + N_STG < n_all:
                    start_chunk(i + N_STG)

        def gemm(s):
            acc_ref[pl.ds(s * SR, SR), :] = lax.dot_general(
                dy_bf[pl.ds(s * SR, SR), :], w_bf[...],
                dimension_numbers=(((1,), (1,)), ((), ())),
                preferred_element_type=jnp.float32,
            )

        def make_xchg(s):
            return pltpu.make_async_remote_copy(
                src_ref=xsend_ref.at[pl.ds(s * SR, SR), :],
                dst_ref=xrecv_ref.at[pl.ds(s * SR, SR), :],
                send_sem=x_sems.at[s, 0],
                recv_sem=x_sems.at[s, 1],
                device_id=xpeer,
                device_id_type=pl.DeviceIdType.MESH,
            )

        def compute_stream(s):
            lo = n_w + s * per_s
            finish_upto(lo, lo + per_s)
            gemm(s)
            xsend_ref[pl.ds(s * SR, SR), :] = acc_ref[
                pl.ds(s * SR, SR), :].astype(jnp.bfloat16)
            x = make_xchg(s)
            x.start()
            return x

        def finalize(s):
            rows = pl.ds(s * SR, SR)
            fin = (
                acc_ref[rows, :] + xrecv_ref[rows, :].astype(jnp.float32)
            ).astype(jnp.bfloat16)
            out_ref[pl.ds(my_y * BR + s * SR, SR),
                    pl.ds(my_z * BC, BC)] = fin
            ring_rows = pl.ds(s * SR + rroff, RR)
            own_stg[s, :, :] = (
                acc_ref[ring_rows, :]
                + xrecv_ref[ring_rows, :].astype(jnp.float32)
            ).astype(jnp.bfloat16)

        def make_cw(s, h):
            if h == 0:
                src = own_stg.at[s]
            elif h < 3:
                src = cwb.at[s, h - 1]
            else:
                src = cwb.at[s, 2, pl.ds(my_x * MS, 2 * MS), :]
            dst = (cwb.at[s, h] if h < 3
                   else cwb.at[s, 3, pl.ds(0, 2 * MS), :])
            return pltpu.make_async_remote_copy(
                src_ref=src, dst_ref=dst,
                send_sem=cw_send.at[s, h], recv_sem=cw_recv.at[s, h],
                device_id=nxt, device_id_type=pl.DeviceIdType.MESH,
            )

        def make_ccw(s, h):
            if h == 0:
                src = own_stg.at[s]
            elif h < 3:
                src = ccwb.at[s, h - 1]
            else:
                src = ccwb.at[s, 2, pl.ds((1 - my_x) * 2 * MS, MS), :]
            dst = (ccwb.at[s, h] if h < 3
                   else ccwb.at[s, 3, pl.ds(0, MS), :])
            return pltpu.make_async_remote_copy(
                src_ref=src, dst_ref=dst,
                send_sem=ccw_send.at[s, h], recv_sem=ccw_recv.at[s, h],
                device_id=prv, device_id_type=pl.DeviceIdType.MESH,
            )

        mirrors = []

        def mirror(s, j, src):
            m = pltpu.make_async_remote_copy(
                src_ref=src, dst_ref=mbuf.at[s, j],
                send_sem=m_send.at[s, j], recv_sem=m_recv.at[s, j],
                device_id=xpeer, device_id_type=pl.DeviceIdType.MESH,
            )
            m.start()
            mirrors.append(m)

        r = [[None] * N_HOP for _ in range(NS)]
        sent = []

        def inject(s):
            finalize(s)
            r[s][0] = (make_cw(s, 0), make_ccw(s, 0))
            r[s][0][0].start()
            r[s][0][1].start()

        def advance(s, h):
            cw, ccw = r[s][h]
            cw.wait_recv()
            ccw.wait_recv()
            sent.append(cw)
            sent.append(ccw)
            if h + 1 < N_HOP:
                r[s][h + 1] = (make_cw(s, h + 1), make_ccw(s, h + 1))
                r[s][h + 1][0].start()
                r[s][h + 1][1].start()
            if h < 3:
                cb = lax.rem(p + N_RING - h - 1, N_RING)
                wb = lax.rem(p + h + 1, N_RING)
                out_ref[blk_rows(cb, s, rroff, RR)] = cwb[s, h, :, :]
                out_ref[blk_rows(wb, s, rroff, RR)] = ccwb[s, h, :, :]
                mirror(s, 2 * h, cwb.at[s, h, pl.ds(mrel, MS), :])
                mirror(s, 2 * h + 1, ccwb.at[s, h, pl.ds(mrel, MS), :])
            else:
                ab = lax.rem(p + N_HOP, N_RING)
                out_ref[blk_rows(ab, s, rroff + my_x * MS, 2 * MS)] = (
                    cwb[s, 3, pl.ds(0, 2 * MS), :])
                out_ref[blk_rows(ab, s, rroff + (1 - my_x) * 2 * MS,
                                 MS)] = ccwb[s, 3, pl.ds(0, MS), :]
                mirror(s, 6, cwb.at[s, 3, pl.ds(my_x * MS, MS), :])

        for j in range(N_STG):
            start_chunk(j)
        finish_upto(0, n_w + per_s)
        gemm(0)

        barrier = pltpu.get_barrier_semaphore()
        for tgt in [xpeer, nxt, prv]:
            pl.semaphore_signal(barrier, inc=1, device_id=tgt,
                                device_id_type=pl.DeviceIdType.MESH)
        pl.semaphore_wait(barrier, 3)

        xsend_ref[pl.ds(0, SR), :] = acc_ref[pl.ds(0, SR), :].astype(
            jnp.bfloat16)
        xchg = [None] * NS
        xchg[0] = make_xchg(0)
        xchg[0].start()

        xchg[1] = compute_stream(1)
        xchg[0].wait_recv()
        sent.append(xchg[0])
        inject(0)
        xchg[2] = compute_stream(2)
        xchg[3] = compute_stream(3)
        xchg[1].wait_recv()
        sent.append(xchg[1])
        inject(1)
        advance(0, 0)
        xchg[2].wait_recv()
        sent.append(xchg[2])
        inject(2)
        advance(1, 0)
        advance(0, 1)
        xchg[3].wait_recv()
        sent.append(xchg[3])
        inject(3)
        advance(2, 0)
        advance(1, 1)
        advance(0, 2)
        advance(3, 0)
        advance(2, 1)
        advance(1, 2)
        advance(0, 3)
        advance(3, 1)
        advance(2, 2)
        advance(1, 3)
        advance(3, 2)
        advance(2, 3)
        advance(3, 3)
        for d in sent:
            d.wait_send()
        for m in mirrors:
            m.wait()
        for s in range(NS):
            for h in range(3):
                cb = lax.rem(p + N_RING - h - 1, N_RING)
                wb = lax.rem(p + h + 1, N_RING)
                out_ref[blk_rows(cb, s, (1 - my_x) * (RR), MS)] = (
                    mbuf[s, 2 * h, :, :])
                out_ref[blk_rows(wb, s, (1 - my_x) * (RR), MS)] = (
                    mbuf[s, 2 * h + 1, :, :])
            ab = lax.rem(p + N_HOP, N_RING)
            out_ref[blk_rows(ab, s, (1 - my_x) * (RR), MS)] = (
                mbuf[s, 6, :, :])

    return pl.pallas_call(
        body,
        out_shape=jax.ShapeDtypeStruct((M, D), jnp.bfloat16),
        in_specs=[
            pl.BlockSpec(memory_space=pl.ANY),
            pl.BlockSpec(memory_space=pl.ANY),
        ],
        out_specs=pl.BlockSpec(memory_space=pltpu.VMEM),
        scratch_shapes=[
            pltpu.VMEM((BR, F), jnp.bfloat16),
            pltpu.VMEM((BC, F), jnp.bfloat16),
            pltpu.VMEM((N_STG, CHUNK, F), jnp.float32),
            pltpu.VMEM((BR, BC), jnp.float32),
            pltpu.VMEM((BR, BC), jnp.bfloat16),
            pltpu.VMEM((BR, BC), jnp.bfloat16),
            pltpu.VMEM((NS, RR, BC), jnp.bfloat16),
            pltpu.VMEM((NS, N_HOP, RR, BC), jnp.bfloat16),
            pltpu.VMEM((NS, N_HOP, RR, BC), jnp.bfloat16),
            pltpu.VMEM((NS, N_MIR, MS, BC), jnp.bfloat16),
            pltpu.SemaphoreType.DMA((N_STG,)),
            pltpu.SemaphoreType.DMA((NS, N_HOP)),
            pltpu.SemaphoreType.DMA((NS, N_HOP)),
            pltpu.SemaphoreType.DMA((NS, N_HOP)),
            pltpu.SemaphoreType.DMA((NS, N_HOP)),
            pltpu.SemaphoreType.DMA((NS, 2)),
            pltpu.SemaphoreType.DMA((NS, N_MIR)),
            pltpu.SemaphoreType.DMA((NS, N_MIR)),
        ],
        compiler_params=pltpu.CompilerParams(
            collective_id=0,
            vmem_limit_bytes=110 * 1024 * 1024,
        ),
    )(dy, W)
